# Initial kernel scaffold; baseline (speedup 1.0000x reference)
import functools

import jax
import jax.numpy as jnp
from jax import lax
from jax.experimental import pallas as pl
from jax.experimental.pallas import tpu as pltpu
from jax.experimental.pallas import tpu_sc as plsc

N_NODES = 10000
D = 128
N_EDGES = 320000

NUM_CORES = 2
NUM_SUBCORES = 16
NW = NUM_CORES * NUM_SUBCORES

G = 128
NCHUNK = -(-N_EDGES // (NW * G))
E_PAD = NW * NCHUNK * G
PADN = 10240
STRIPE = PADN // NUM_SUBCORES


def _sc_agg_body(x_hbm, src_hbm, dst_hbm, out_hbm, src_v, dst_v, rows_v,
                 acc, sem):
  c = lax.axis_index("c")
  s = lax.axis_index("s")
  wid = s * NUM_CORES + c

  zz = jnp.zeros((16,), jnp.float32)

  def zero_row(i, _):
    for j in range(D // 16):
      rows_v[i, pl.ds(j * 16, 16)] = zz
    return 0

  lax.fori_loop(0, G, zero_row, 0)
  for k in range(STRIPE // G):
    pltpu.sync_copy(rows_v, acc.at[pl.ds(s * STRIPE + k * G, G)])
  plsc.subcore_barrier()

  pltpu.sync_copy(src_hbm.at[wid], src_v)
  pltpu.sync_copy(dst_hbm.at[wid], dst_v)

  def chunk(j, _):
    pltpu.async_copy(x_hbm.at[src_v.at[j]], rows_v, sem).wait()
    pltpu.sync_copy(rows_v, acc.at[dst_v.at[j]], add=True)
    return 0

  lax.fori_loop(0, NCHUNK, chunk, 0)
  plsc.subcore_barrier()

  for k in range(STRIPE // G):
    base = s * STRIPE + k * G
    pltpu.sync_copy(acc.at[pl.ds(base, G)], rows_v)
    pltpu.sync_copy(rows_v, out_hbm.at[c, pl.ds(base, G)])


_sc_agg = functools.partial(
    pl.kernel,
    out_type=jax.ShapeDtypeStruct((NUM_CORES, PADN, D), jnp.float32),
    mesh=plsc.VectorSubcoreMesh(core_axis_name="c", subcore_axis_name="s"),
    scratch_types=[
        pltpu.VMEM((NCHUNK, G), jnp.int32),
        pltpu.VMEM((NCHUNK, G), jnp.int32),
        pltpu.VMEM((G, D), jnp.float32),
        pltpu.VMEM_SHARED((PADN, D), jnp.float32),
        pltpu.SemaphoreType.DMA,
    ],
)(_sc_agg_body)


def _tc_epilogue_body(p_ref, w_ref, o_ref):
  h = p_ref[0] + p_ref[1]
  h = jnp.dot(h, w_ref[...], preferred_element_type=jnp.float32)
  o_ref[...] = jnp.where(h > 0, h, jnp.expm1(h))


_BR = 1250


@jax.jit
def kernel(x, edge_index, W):
  src = edge_index[0].astype(jnp.int32)
  dst = edge_index[1].astype(jnp.int32)
  pad = E_PAD - N_EDGES
  src_p = jnp.concatenate([src, jnp.zeros((pad,), jnp.int32)])
  dst_p = jnp.concatenate([dst, jnp.full((pad,), N_NODES, jnp.int32)])
  src_p = src_p.reshape(NW, NCHUNK, G)
  dst_p = dst_p.reshape(NW, NCHUNK, G)

  partials = _sc_agg(x, src_p, dst_p)

  out = pl.pallas_call(
      _tc_epilogue_body,
      grid=(N_NODES // _BR,),
      in_specs=[
          pl.BlockSpec((NUM_CORES, _BR, D), lambda i: (0, i, 0)),
          pl.BlockSpec((D, D), lambda i: (0, 0)),
      ],
      out_specs=pl.BlockSpec((_BR, D), lambda i: (i, 0)),
      out_shape=jax.ShapeDtypeStruct((N_NODES, D), jnp.float32),
  )(partials, W)
  return out


# speedup vs baseline: 4.9293x; 4.9293x over previous
import functools

import jax
import jax.numpy as jnp
from jax import lax
from jax.experimental import pallas as pl
from jax.experimental.pallas import tpu as pltpu
from jax.experimental.pallas import tpu_sc as plsc

N_NODES = 10000
D = 128
N_EDGES = 320000

NUM_CORES = 2
NUM_SUBCORES = 16
NW = NUM_CORES * NUM_SUBCORES

G = 128
NCHUNK = -(-N_EDGES // (NW * G))
E_PAD = NW * NCHUNK * G
PADN = 10240
STRIPE = PADN // NUM_SUBCORES


def _sc_agg_body(x_hbm, src_hbm, dst_hbm, out_hbm, src_v, dst_v, rows_v,
                 acc, sem):
  c = lax.axis_index("c")
  s = lax.axis_index("s")
  wid = s * NUM_CORES + c

  zz = jnp.zeros((16,), jnp.float32)

  def zero_row(i, _):
    for j in range(D // 16):
      rows_v[i, pl.ds(j * 16, 16)] = zz
    return 0

  lax.fori_loop(0, G, zero_row, 0)
  for k in range(STRIPE // G):
    pltpu.sync_copy(rows_v, acc.at[pl.ds(s * STRIPE + k * G, G)])
  plsc.subcore_barrier()

  pltpu.sync_copy(src_hbm.at[wid], src_v)
  pltpu.sync_copy(dst_hbm.at[wid], dst_v)

  def chunk(j, _):
    pltpu.async_copy(x_hbm.at[src_v.at[j]], rows_v, sem).wait()
    pltpu.sync_copy(rows_v, acc.at[dst_v.at[j]], add=True)
    return 0

  lax.fori_loop(0, NCHUNK, chunk, 0)
  plsc.subcore_barrier()

  for k in range(STRIPE // G):
    base = s * STRIPE + k * G
    pltpu.sync_copy(acc.at[pl.ds(base, G)], rows_v)
    pltpu.sync_copy(rows_v, out_hbm.at[c, pl.ds(base, G)])


_sc_agg = functools.partial(
    pl.kernel,
    out_type=jax.ShapeDtypeStruct((NUM_CORES, PADN, D), jnp.float32),
    mesh=plsc.VectorSubcoreMesh(core_axis_name="c", subcore_axis_name="s"),
    scratch_types=[
        pltpu.VMEM((NCHUNK, G), jnp.int32),
        pltpu.VMEM((NCHUNK, G), jnp.int32),
        pltpu.VMEM((G, D), jnp.float32),
        pltpu.VMEM_SHARED((PADN, D), jnp.float32),
        pltpu.SemaphoreType.DMA,
    ],
)(_sc_agg_body)


def _tc_epilogue_body(p_ref, w_ref, o_ref):
  h = p_ref[0] + p_ref[1]
  h = jnp.dot(h, w_ref[...], preferred_element_type=jnp.float32)
  o_ref[...] = jnp.where(h > 0, h, jnp.exp(jnp.minimum(h, 0.0)) - 1.0)


_BR = 1000


@jax.jit
def kernel(x, edge_index, W):
  src = edge_index[0].astype(jnp.int32)
  dst = edge_index[1].astype(jnp.int32)
  pad = E_PAD - N_EDGES
  src_p = jnp.concatenate([src, jnp.zeros((pad,), jnp.int32)])
  dst_p = jnp.concatenate([dst, jnp.full((pad,), N_NODES, jnp.int32)])
  src_p = src_p.reshape(NW, NCHUNK, G)
  dst_p = dst_p.reshape(NW, NCHUNK, G)

  partials = _sc_agg(x, src_p, dst_p)

  out = pl.pallas_call(
      _tc_epilogue_body,
      grid=(N_NODES // _BR,),
      in_specs=[
          pl.BlockSpec((NUM_CORES, _BR, D), lambda i: (0, i, 0)),
          pl.BlockSpec((D, D), lambda i: (0, 0)),
      ],
      out_specs=pl.BlockSpec((_BR, D), lambda i: (i, 0)),
      out_shape=jax.ShapeDtypeStruct((N_NODES, D), jnp.float32),
  )(partials, W)
  return out
